# Initial kernel scaffold; baseline (speedup 1.0000x reference)
import jax
import jax.numpy as jnp
from jax.experimental import pallas as pl


def kernel(x, edge_index0, edge_index1, edge_index2, edge_index3, edge_index4, edge_index5, edge_attr0, edge_attr1, edge_attr2, edge_attr3, edge_attr4, edge_attr5, P01, P12, P23, P34, P45, P56, W1, W2, W3, W4, W5, W6, R1, R2, R3, R4, R5, R6, B1, B2, B3, B4, B5, B6):
    raise NotImplementedError("write your pallas kernel here")


# speedup vs baseline: 1.1275x; 1.1275x over previous
import functools

import jax
import jax.numpy as jnp
from jax.experimental import pallas as pl

K = 5
K3 = 125
_NF = [2, 8, 16, 32, 64, 128, 128]
_NNODES = [10000, 1250, 640, 320, 160, 80, 40]


def _edge_prep(edge_attr, dst):
    p = jnp.clip(edge_attr, 0.0, 1.0) * (K - 1)
    bot = jnp.clip(jnp.floor(p), 0.0, float(K - 2))
    frac = p - bot
    boti = bot.astype(jnp.int32)
    bs, segs = [], []
    for c0 in range(2):
        for c1 in range(2):
            for c2 in range(2):
                b0 = frac[:, 0] if c0 else 1.0 - frac[:, 0]
                b1 = frac[:, 1] if c1 else 1.0 - frac[:, 1]
                b2 = frac[:, 2] if c2 else 1.0 - frac[:, 2]
                wi = (boti[:, 0] + c0) + (boti[:, 1] + c1) * K + (boti[:, 2] + c2) * (K * K)
                bs.append(b0 * b1 * b2)
                segs.append(dst * K3 + wi)
    return jnp.stack(bs), jnp.stack(segs)


def _scatter_A(x, src, b, seg, num_nodes):
    fin = x.shape[1]
    x_src = x[src]
    msg = jnp.concatenate([b[:, :, None] * x_src[None, :, :],
                           b[:, :, None]], axis=-1)
    A = jnp.zeros((num_nodes * K3, fin + 1), dtype=x.dtype)
    A = A.at[seg.reshape(-1)].add(msg.reshape(-1, fin + 1))
    return A


def _dense_tail_body(a_ref, wext_ref, x_ref, r_ref, bias_ref, p_ref, acc_ref,
                     *, fout, last_level):
    i = pl.program_id(0)

    @pl.when(i == 0)
    def _init():
        acc_ref[...] = jnp.zeros_like(acc_ref)

    z = jnp.dot(a_ref[...], wext_ref[...], preferred_element_type=jnp.float32)
    conv = z[:, :fout]
    deg = z[:, fout:fout + 1]
    h = conv / jnp.maximum(deg, 1.0)
    h = h + jnp.dot(x_ref[...], r_ref[...], preferred_element_type=jnp.float32)
    h = h + bias_ref[...]
    h = jnp.where(h > 0, h, jnp.exp(jnp.minimum(h, 0.0)) - 1.0)
    pooled = jnp.dot(p_ref[...].T, h, preferred_element_type=jnp.float32)
    if last_level:
        acc_ref[...] = jnp.max(pooled, axis=0, keepdims=True)
    else:
        acc_ref[...] += pooled


def _dense_tail(A, W, x, R, bias, P, num_nodes, n_next, block_n, last_level):
    fin = x.shape[1]
    fout = R.shape[1]
    W_ext = jnp.concatenate([W, jnp.zeros((K3, fin, 1), W.dtype)], axis=2)
    brow = jnp.zeros((K3, 1, fout + 1), W.dtype).at[:, 0, fout].set(1.0)
    W_ext = jnp.concatenate([W_ext, brow], axis=1).reshape(K3 * (fin + 1), fout + 1)

    A2 = A.reshape(num_nodes, K3 * (fin + 1))
    grid = (num_nodes // block_n,)
    out_rows = 1 if last_level else n_next
    return pl.pallas_call(
        functools.partial(_dense_tail_body, fout=fout, last_level=last_level),
        grid=grid,
        in_specs=[
            pl.BlockSpec((block_n, K3 * (fin + 1)), lambda i: (i, 0)),
            pl.BlockSpec((K3 * (fin + 1), fout + 1), lambda i: (0, 0)),
            pl.BlockSpec((block_n, fin), lambda i: (i, 0)),
            pl.BlockSpec((fin, fout), lambda i: (0, 0)),
            pl.BlockSpec((1, fout), lambda i: (0, 0)),
            pl.BlockSpec((block_n, n_next), lambda i: (i, 0)),
        ],
        out_specs=pl.BlockSpec((out_rows, fout), lambda i: (0, 0)),
        out_shape=jax.ShapeDtypeStruct((out_rows, fout), jnp.float32),
    )(A2, W_ext, x, R, bias.reshape(1, fout), P)


def kernel(x, edge_index0, edge_index1, edge_index2, edge_index3, edge_index4, edge_index5, edge_attr0, edge_attr1, edge_attr2, edge_attr3, edge_attr4, edge_attr5, P01, P12, P23, P34, P45, P56, W1, W2, W3, W4, W5, W6, R1, R2, R3, R4, R5, R6, B1, B2, B3, B4, B5, B6):
    EI = [edge_index0, edge_index1, edge_index2, edge_index3, edge_index4, edge_index5]
    EA = [edge_attr0, edge_attr1, edge_attr2, edge_attr3, edge_attr4, edge_attr5]
    Ps = [P01, P12, P23, P34, P45, P56]
    Ws = [W1, W2, W3, W4, W5, W6]
    Rs = [R1, R2, R3, R4, R5, R6]
    Bs = [B1, B2, B3, B4, B5, B6]
    BLOCK = [1000, 1250, 640, 320, 160, 80]

    h = x
    for l in range(6):
        src, dst = EI[l][0], EI[l][1]
        b, seg = _edge_prep(EA[l], dst)
        A = _scatter_A(h, src, b, seg, _NNODES[l])
        h = _dense_tail(A, Ws[l], h, Rs[l], Bs[l], Ps[l],
                        _NNODES[l], _NNODES[l + 1], BLOCK[l],
                        last_level=(l == 5))
    return h
